# baseline (speedup 1.0000x reference)
import functools

import jax
import jax.numpy as jnp
from jax import lax
from jax.experimental import pallas as pl
from jax.experimental.pallas import tpu as pltpu
from jax.experimental.pallas import tpu_sc as plsc

NC = 2
NS = 16
LANES = 16
B = 80
SR = 80

_HIGH = lax.Precision.HIGHEST


def _sc_degree(col3, n):
    w, ngrp, grp, _ = col3.shape
    nrch = n // SR
    rounds = (nrch + NS - 1) // NS

    mesh = plsc.VectorSubcoreMesh(core_axis_name="c", subcore_axis_name="s")

    @functools.partial(
        pl.kernel,
        out_type=jax.ShapeDtypeStruct((NC, n, LANES), jnp.float32),
        mesh=mesh,
        scratch_types=[
            pltpu.VMEM((ngrp, grp, B), jnp.int32),
            pltpu.VMEM((B, LANES), jnp.float32),
            pltpu.VMEM((SR, LANES), jnp.float32),
            pltpu.VMEM_SHARED((n, LANES), jnp.float32),
            pltpu.SemaphoreType.DMA,
        ],
    )
    def deg_kernel(col_hbm, out_hbm, cidx_v, ones_v, stage_v, acc_sh, sem):
        cid = lax.axis_index("c")
        sid = lax.axis_index("s")
        wid = sid * NC + cid

        idx_cp = pltpu.async_copy(col_hbm.at[wid], cidx_v, sem)

        def fill_ones(i, _):
            ones_v[i, :] = jnp.ones((LANES,), jnp.float32)
            return 0
        lax.fori_loop(0, B, fill_ones, 0)

        init = jnp.where(cid == 0, 1.0, 0.0).astype(jnp.float32)

        def fill_stage(i, _):
            stage_v[i, :] = jnp.broadcast_to(init, (LANES,))
            return 0
        lax.fori_loop(0, SR, fill_stage, 0)

        def init_cp(t, _):
            ch = sid + t * NS
            @pl.when(ch < nrch)
            def _():
                pltpu.sync_copy(stage_v, acc_sh.at[pl.ds(ch * SR, SR)])
            return 0
        lax.fori_loop(0, rounds, init_cp, 0)
        idx_cp.wait()
        plsc.subcore_barrier()

        def body(gi, _):
            def inner(gj, _):
                pltpu.sync_copy(ones_v, acc_sh.at[cidx_v.at[gi, gj]],
                                add=True)
                return 0
            return lax.fori_loop(0, grp, inner, 0)
        lax.fori_loop(0, ngrp, body, 0)

        plsc.subcore_barrier()

        def wb(t, _):
            ch = sid + t * NS
            @pl.when(ch < nrch)
            def _():
                pltpu.sync_copy(acc_sh.at[pl.ds(ch * SR, SR)], stage_v)
                pltpu.sync_copy(stage_v, out_hbm.at[cid, pl.ds(ch * SR, SR)])
            return 0
        lax.fori_loop(0, rounds, wb, 0)

    return deg_kernel(col3)


def _sc_scatter(y, row3, col3):
    n, h = y.shape
    w, ngrp, grp, _ = row3.shape
    pairs = (grp - 1) // 2
    nrch = n // SR
    rounds = (nrch + NS - 1) // NS

    mesh = plsc.VectorSubcoreMesh(core_axis_name="c", subcore_axis_name="s")

    @functools.partial(
        pl.kernel,
        out_type=jax.ShapeDtypeStruct((NC, n, h), jnp.float32),
        mesh=mesh,
        scratch_types=[
            pltpu.VMEM((2, grp, B), jnp.int32),
            pltpu.VMEM((2, grp, B), jnp.int32),
            pltpu.VMEM((B, h), jnp.float32),
            pltpu.VMEM((B, h), jnp.float32),
            pltpu.VMEM_SHARED((n, h), jnp.float32),
            pltpu.SemaphoreType.DMA,
            pltpu.SemaphoreType.DMA,
            pltpu.SemaphoreType.DMA,
            pltpu.SemaphoreType.DMA,
            pltpu.SemaphoreType.DMA,
            pltpu.SemaphoreType.DMA,
        ],
    )
    def scat_kernel(y_hbm, row_hbm, col_hbm, out_hbm,
                    ridx_v, cidx_v, rows0_v, rows1_v, acc_sh,
                    sem0, sem1, sem_s0, sem_s1, sem_ri, sem_ci):
        stage_v = rows0_v
        cid = lax.axis_index("c")
        sid = lax.axis_index("s")
        wid = sid * NC + cid

        pltpu.async_copy(row_hbm.at[wid, 0], ridx_v.at[0], sem_ri)
        pltpu.async_copy(col_hbm.at[wid, 0], cidx_v.at[0], sem_ci)

        def zrow(i, _):
            def zcol(j, _):
                stage_v[i, pl.ds(j * LANES, LANES)] = (
                    jnp.zeros((LANES,), jnp.float32))
                return 0
            return lax.fori_loop(0, h // LANES, zcol, 0)

        @pl.when(cid != 0)
        def _():
            lax.fori_loop(0, SR, zrow, 0)

        def init_cp(t, _):
            ch = sid + t * NS
            @pl.when(ch < nrch)
            def _():
                @pl.when(cid == 0)
                def _():
                    pltpu.sync_copy(y_hbm.at[pl.ds(ch * SR, SR)], stage_v)
                pltpu.sync_copy(stage_v, acc_sh.at[pl.ds(ch * SR, SR)])
            return 0
        lax.fori_loop(0, rounds, init_cp, 0)
        plsc.subcore_barrier()

        for gi in range(ngrp):
            ridx = ridx_v.at[gi % 2]
            cidx = cidx_v.at[gi % 2]
            pltpu.make_async_copy(row_hbm.at[wid, gi], ridx, sem_ri).wait()
            pltpu.make_async_copy(col_hbm.at[wid, gi], cidx, sem_ci).wait()
            if gi + 1 < ngrp:
                pltpu.async_copy(row_hbm.at[wid, gi + 1],
                                 ridx_v.at[(gi + 1) % 2], sem_ri)
                pltpu.async_copy(col_hbm.at[wid, gi + 1],
                                 cidx_v.at[(gi + 1) % 2], sem_ci)

            def g_wait(g, buf, sem):
                pltpu.make_async_copy(y_hbm.at[ridx.at[g]], buf, sem).wait()

            def s_start(g, buf, sem):
                pltpu.async_copy(buf, acc_sh.at[cidx.at[g]], sem, add=True)

            def s_wait(g, buf, sem):
                pltpu.make_async_copy(buf, acc_sh.at[cidx.at[g]], sem).wait()

            def g_start(g, buf, sem):
                pltpu.async_copy(y_hbm.at[ridx.at[g]], buf, sem)

            g_start(0, rows0_v, sem0)
            g_wait(0, rows0_v, sem0)
            s_start(0, rows0_v, sem_s0)
            g_start(1, rows1_v, sem1)

            def pair(t, _):
                g = 2 * t + 1
                g_wait(g, rows1_v, sem1)
                s_start(g, rows1_v, sem_s1)
                s_wait(g - 1, rows0_v, sem_s0)
                g_start(g + 1, rows0_v, sem0)
                g_wait(g + 1, rows0_v, sem0)
                s_start(g + 1, rows0_v, sem_s0)
                s_wait(g, rows1_v, sem_s1)
                g_start(g + 2, rows1_v, sem1)
                return 0
            lax.fori_loop(0, (grp - 3) // 2, pair, 0)

            gl = grp - 2
            g_wait(gl, rows1_v, sem1)
            s_start(gl, rows1_v, sem_s1)
            s_wait(gl - 1, rows0_v, sem_s0)
            g_start(gl + 1, rows0_v, sem0)
            g_wait(gl + 1, rows0_v, sem0)
            s_start(gl + 1, rows0_v, sem_s0)
            s_wait(gl, rows1_v, sem_s1)
            s_wait(gl + 1, rows0_v, sem_s0)

        plsc.subcore_barrier()

        def wb(t, _):
            ch = sid + t * NS
            @pl.when(ch < nrch)
            def _():
                pltpu.sync_copy(acc_sh.at[pl.ds(ch * SR, SR)], stage_v)
                pltpu.sync_copy(stage_v, out_hbm.at[cid, pl.ds(ch * SR, SR)])
            return 0
        lax.fori_loop(0, rounds, wb, 0)

    return scat_kernel(y, row3, col3)


_NBLK = 10


def _dinv_from(deg_ref):
    deg = jnp.sum(deg_ref[0] + deg_ref[1], axis=1) * (1.0 / LANES)
    return 1.0 / jnp.sqrt(deg)


def _tc_prescale_matmul(x, wmat, deg2):
    n, d = x.shape
    h = wmat.shape[1]
    r = n // _NBLK

    def body(x_ref, w_ref, deg_ref, y_ref):
        dinv = _dinv_from(deg_ref)
        y_ref[...] = jnp.dot(x_ref[...], w_ref[...],
                             preferred_element_type=jnp.float32,
                             precision=_HIGH) * dinv[:, None]

    return pl.pallas_call(
        body,
        grid=(_NBLK,),
        in_specs=[
            pl.BlockSpec((r, d), lambda i: (i, 0)),
            pl.BlockSpec((d, h), lambda i: (0, 0)),
            pl.BlockSpec((NC, r, LANES), lambda i: (0, i, 0)),
        ],
        out_specs=pl.BlockSpec((r, h), lambda i: (i, 0)),
        out_shape=jax.ShapeDtypeStruct((n, h), jnp.float32),
    )(x, wmat, deg2)


def _tc_post_stats(s2, deg2, b):
    _, n, h = s2.shape
    r = n // _NBLK

    def body(s_ref, deg_ref, b_ref, h_ref, st_ref):
        i = pl.program_id(0)
        dinv = _dinv_from(deg_ref)
        hp = (s_ref[0] + s_ref[1]) * dinv[:, None] + b_ref[...]
        h_ref[...] = hp

        @pl.when(i == 0)
        def _():
            st_ref[...] = jnp.zeros_like(st_ref)
        st_ref[0:1, :] += jnp.sum(hp, axis=0, keepdims=True)
        st_ref[1:2, :] += jnp.sum(hp * hp, axis=0, keepdims=True)

    return pl.pallas_call(
        body,
        grid=(_NBLK,),
        in_specs=[
            pl.BlockSpec((NC, r, h), lambda i: (0, i, 0)),
            pl.BlockSpec((NC, r, LANES), lambda i: (0, i, 0)),
            pl.BlockSpec((1, h), lambda i: (0, 0)),
        ],
        out_specs=[
            pl.BlockSpec((r, h), lambda i: (i, 0)),
            pl.BlockSpec((2, h), lambda i: (0, 0)),
        ],
        out_shape=[
            jax.ShapeDtypeStruct((n, h), jnp.float32),
            jax.ShapeDtypeStruct((2, h), jnp.float32),
        ],
    )(s2, deg2, b)


def _tc_bn_relu_matmul(h_pre, stats, gamma, beta, wmat, deg2):
    n, h = h_pre.shape
    h2 = wmat.shape[1]
    r = n // _NBLK
    inv_n = 1.0 / n

    def body(h_ref, st_ref, g_ref, be_ref, w_ref, deg_ref, y_ref):
        dinv = _dinv_from(deg_ref)
        mu = st_ref[0:1, :] * inv_n
        var = st_ref[1:2, :] * inv_n - mu * mu
        hn = (h_ref[...] - mu) / jnp.sqrt(var + 1e-5) * g_ref[...] + be_ref[...]
        hr = jnp.maximum(hn, 0.0)
        y_ref[...] = jnp.dot(hr, w_ref[...],
                             preferred_element_type=jnp.float32,
                             precision=_HIGH) * dinv[:, None]

    return pl.pallas_call(
        body,
        grid=(_NBLK,),
        in_specs=[
            pl.BlockSpec((r, h), lambda i: (i, 0)),
            pl.BlockSpec((2, h), lambda i: (0, 0)),
            pl.BlockSpec((1, h), lambda i: (0, 0)),
            pl.BlockSpec((1, h), lambda i: (0, 0)),
            pl.BlockSpec((h, h2), lambda i: (0, 0)),
            pl.BlockSpec((NC, r, LANES), lambda i: (0, i, 0)),
        ],
        out_specs=pl.BlockSpec((r, h2), lambda i: (i, 0)),
        out_shape=jax.ShapeDtypeStruct((n, h2), jnp.float32),
    )(h_pre, stats, gamma, beta, wmat, deg2)


def _tc_bn_relu_pool_head(h_pre, stats, gamma, beta, batch3d, wf, bf, g):
    n, h = h_pre.shape
    c = wf.shape[1]
    r = n // _NBLK
    inv_n = 1.0 / n

    def body(h_ref, st_ref, g_ref, be_ref, bat_ref, wf_ref, bf_ref, o_ref,
             pooled_ref, counts_ref):
        i = pl.program_id(0)
        mu = st_ref[0:1, :] * inv_n
        var = st_ref[1:2, :] * inv_n - mu * mu
        hn = (h_ref[...] - mu) / jnp.sqrt(var + 1e-5) * g_ref[...] + be_ref[...]
        hr = jnp.maximum(hn, 0.0)

        gids = bat_ref[0]
        gi = lax.broadcasted_iota(jnp.int32, (g, r), 0)
        m = (gi == gids).astype(jnp.float32)

        @pl.when(i == 0)
        def _():
            pooled_ref[...] = jnp.zeros_like(pooled_ref)
            counts_ref[...] = jnp.zeros_like(counts_ref)

        pooled_ref[...] += jnp.dot(m, hr, preferred_element_type=jnp.float32,
                                   precision=_HIGH)
        counts_ref[...] += jnp.sum(m, axis=1, keepdims=True)

        @pl.when(i == _NBLK - 1)
        def _():
            pooled = pooled_ref[...] / jnp.maximum(counts_ref[...], 1.0)
            o_ref[...] = jnp.dot(pooled, wf_ref[...],
                                 preferred_element_type=jnp.float32,
                                 precision=_HIGH) + bf_ref[...]

    return pl.pallas_call(
        body,
        grid=(_NBLK,),
        in_specs=[
            pl.BlockSpec((r, h), lambda i: (i, 0)),
            pl.BlockSpec((2, h), lambda i: (0, 0)),
            pl.BlockSpec((1, h), lambda i: (0, 0)),
            pl.BlockSpec((1, h), lambda i: (0, 0)),
            pl.BlockSpec((1, 1, r), lambda i: (i, 0, 0)),
            pl.BlockSpec((h, c), lambda i: (0, 0)),
            pl.BlockSpec((1, c), lambda i: (0, 0)),
        ],
        out_specs=pl.BlockSpec((g, c), lambda i: (0, 0)),
        out_shape=jax.ShapeDtypeStruct((g, c), jnp.float32),
        scratch_shapes=[
            pltpu.VMEM((g, h), jnp.float32),
            pltpu.VMEM((g, 1), jnp.float32),
        ],
    )(h_pre, stats, gamma, beta, batch3d, wf, bf)


def kernel(x, edge_index, batch, W1, b1, gamma1, beta1, W2, b2, gamma2,
           beta2, Wf, bf):
    n, d = x.shape
    h = W1.shape[1]
    c = Wf.shape[1]
    g = 64

    e = edge_index.shape[1]
    w = NC * NS
    ngrp = 5
    grp = e // (w * B * ngrp)
    row3 = edge_index[0].reshape(w, ngrp, grp, B)
    col3 = edge_index[1].reshape(w, ngrp, grp, B)

    deg2 = _sc_degree(col3, n)
    y1 = _tc_prescale_matmul(x, W1, deg2)
    s1 = _sc_scatter(y1, row3, col3)
    h1, st1 = _tc_post_stats(s1, deg2, b1.reshape(1, h))
    y2 = _tc_bn_relu_matmul(h1, st1, gamma1.reshape(1, h), beta1.reshape(1, h),
                            W2, deg2)
    s2 = _sc_scatter(y2, row3, col3)
    h2, st2 = _tc_post_stats(s2, deg2, b2.reshape(1, h))
    out = _tc_bn_relu_pool_head(h2, st2, gamma2.reshape(1, h),
                                beta2.reshape(1, h),
                                batch.reshape(_NBLK, 1, n // _NBLK),
                                Wf, bf.reshape(1, c), g)
    return out


# speedup vs baseline: 1.2412x; 1.2412x over previous
import functools

import jax
import jax.numpy as jnp
from jax import lax
from jax.experimental import pallas as pl
from jax.experimental.pallas import tpu as pltpu
from jax.experimental.pallas import tpu_sc as plsc

NC = 2
NS = 16
LANES = 16
B = 125
SR = 80

_HIGH = lax.Precision.HIGHEST


def _sc_degree(col3, n):
    w, ngrp, grp, _ = col3.shape
    nrch = n // SR
    rounds = (nrch + NS - 1) // NS

    mesh = plsc.VectorSubcoreMesh(core_axis_name="c", subcore_axis_name="s")

    @functools.partial(
        pl.kernel,
        out_type=jax.ShapeDtypeStruct((NC, n, LANES), jnp.float32),
        mesh=mesh,
        scratch_types=[
            pltpu.VMEM((ngrp, grp, B), jnp.int32),
            pltpu.VMEM((B, LANES), jnp.float32),
            pltpu.VMEM((SR, LANES), jnp.float32),
            pltpu.VMEM_SHARED((n, LANES), jnp.float32),
            pltpu.SemaphoreType.DMA,
            pltpu.SemaphoreType.DMA,
        ],
    )
    def deg_kernel(col_hbm, out_hbm, cidx_v, ones_v, stage_v, acc_sh, sem,
                   sem_b):
        cid = lax.axis_index("c")
        sid = lax.axis_index("s")
        wid = sid * NC + cid

        idx_cp = pltpu.async_copy(col_hbm.at[wid], cidx_v, sem)

        def fill_ones(i, _):
            ones_v[i, :] = jnp.ones((LANES,), jnp.float32)
            return 0
        lax.fori_loop(0, B, fill_ones, 0)

        init = jnp.where(cid == 0, 1.0, 0.0).astype(jnp.float32)

        def fill_stage(i, _):
            stage_v[i, :] = jnp.broadcast_to(init, (LANES,))
            return 0
        lax.fori_loop(0, SR, fill_stage, 0)

        def init_cp(t, _):
            ch = sid + t * NS
            @pl.when(ch < nrch)
            def _():
                pltpu.sync_copy(stage_v, acc_sh.at[pl.ds(ch * SR, SR)])
            return 0
        lax.fori_loop(0, rounds, init_cp, 0)
        idx_cp.wait()
        plsc.subcore_barrier()

        def fire(gi, gj, s):
            pltpu.async_copy(ones_v, acc_sh.at[cidx_v.at[gi, gj]], s,
                             add=True)

        def drain(gi, gj, s):
            pltpu.make_async_copy(ones_v, acc_sh.at[cidx_v.at[gi, gj]],
                                  s).wait()

        for gi in range(ngrp):
            fire(gi, 0, sem)
            fire(gi, 1, sem_b)

            def body(t, _):
                gj = 2 * t + 2
                fire(gi, gj, sem)
                drain(gi, gj - 2, sem)
                fire(gi, gj + 1, sem_b)
                drain(gi, gj - 1, sem_b)
                return 0
            lax.fori_loop(0, (grp - 2) // 2, body, 0)
            drain(gi, grp - 2, sem)
            drain(gi, grp - 1, sem_b)

        plsc.subcore_barrier()

        def wb(t, _):
            ch = sid + t * NS
            @pl.when(ch < nrch)
            def _():
                pltpu.sync_copy(acc_sh.at[pl.ds(ch * SR, SR)], stage_v)
                pltpu.sync_copy(stage_v, out_hbm.at[cid, pl.ds(ch * SR, SR)])
            return 0
        lax.fori_loop(0, rounds, wb, 0)

    return deg_kernel(col3)


def _sc_scatter(y, row3, col3):
    n, h = y.shape
    w, ngrp, grp, _ = row3.shape
    pairs = (grp - 1) // 2 if grp % 2 == 1 else (grp - 2) // 2
    nrch = n // SR
    rounds = (nrch + NS - 1) // NS

    mesh = plsc.VectorSubcoreMesh(core_axis_name="c", subcore_axis_name="s")

    @functools.partial(
        pl.kernel,
        out_type=jax.ShapeDtypeStruct((NC, n, h), jnp.float32),
        mesh=mesh,
        scratch_types=[
            pltpu.VMEM((2, grp, B), jnp.int32),
            pltpu.VMEM((2, grp, B), jnp.int32),
            pltpu.VMEM((B, h), jnp.float32),
            pltpu.VMEM((B, h), jnp.float32),
            pltpu.VMEM_SHARED((n, h), jnp.float32),
            pltpu.SemaphoreType.DMA,
            pltpu.SemaphoreType.DMA,
            pltpu.SemaphoreType.DMA,
            pltpu.SemaphoreType.DMA,
        ],
    )
    def scat_kernel(y_hbm, row_hbm, col_hbm, out_hbm,
                    ridx_v, cidx_v, rows0_v, rows1_v, acc_sh,
                    sem0, sem1, sem_ri, sem_ci):
        stage_v = rows0_v.at[pl.ds(0, SR)]
        cid = lax.axis_index("c")
        sid = lax.axis_index("s")
        wid = sid * NC + cid

        pltpu.async_copy(row_hbm.at[wid, 0], ridx_v.at[0], sem_ri)
        pltpu.async_copy(col_hbm.at[wid, 0], cidx_v.at[0], sem_ci)

        def zrow(i, _):
            def zcol(j, _):
                stage_v[i, pl.ds(j * LANES, LANES)] = (
                    jnp.zeros((LANES,), jnp.float32))
                return 0
            return lax.fori_loop(0, h // LANES, zcol, 0)

        @pl.when(cid != 0)
        def _():
            lax.fori_loop(0, SR, zrow, 0)

        def init_cp(t, _):
            ch = sid + t * NS
            @pl.when(ch < nrch)
            def _():
                @pl.when(cid == 0)
                def _():
                    pltpu.sync_copy(y_hbm.at[pl.ds(ch * SR, SR)], stage_v)
                pltpu.sync_copy(stage_v, acc_sh.at[pl.ds(ch * SR, SR)])
            return 0
        lax.fori_loop(0, rounds, init_cp, 0)
        plsc.subcore_barrier()

        for gi in range(ngrp):
            ridx = ridx_v.at[gi % 2]
            cidx = cidx_v.at[gi % 2]
            pltpu.make_async_copy(row_hbm.at[wid, gi], ridx, sem_ri).wait()
            pltpu.make_async_copy(col_hbm.at[wid, gi], cidx, sem_ci).wait()
            if gi + 1 < ngrp:
                pltpu.async_copy(row_hbm.at[wid, gi + 1],
                                 ridx_v.at[(gi + 1) % 2], sem_ri)
                pltpu.async_copy(col_hbm.at[wid, gi + 1],
                                 cidx_v.at[(gi + 1) % 2], sem_ci)

            pltpu.async_copy(y_hbm.at[ridx.at[0]], rows0_v, sem0)

            def pair(t, _):
                g = 1 + 2 * t
                pltpu.async_copy(y_hbm.at[ridx.at[g]], rows1_v, sem1)
                pltpu.make_async_copy(y_hbm.at[ridx.at[g - 1]], rows0_v,
                                      sem0).wait()
                pltpu.sync_copy(rows0_v, acc_sh.at[cidx.at[g - 1]], add=True)
                pltpu.async_copy(y_hbm.at[ridx.at[g + 1]], rows0_v, sem0)
                pltpu.make_async_copy(y_hbm.at[ridx.at[g]], rows1_v,
                                      sem1).wait()
                pltpu.sync_copy(rows1_v, acc_sh.at[cidx.at[g]], add=True)
                return 0
            lax.fori_loop(0, pairs, pair, 0)

            last = pairs * 2
            if grp % 2 == 0:
                pltpu.async_copy(y_hbm.at[ridx.at[last + 1]], rows1_v, sem1)
            pltpu.make_async_copy(y_hbm.at[ridx.at[last]], rows0_v,
                                  sem0).wait()
            pltpu.sync_copy(rows0_v, acc_sh.at[cidx.at[last]], add=True)
            if grp % 2 == 0:
                pltpu.make_async_copy(y_hbm.at[ridx.at[last + 1]], rows1_v,
                                      sem1).wait()
                pltpu.sync_copy(rows1_v, acc_sh.at[cidx.at[last + 1]],
                                add=True)

        plsc.subcore_barrier()

        def wb(t, _):
            ch = sid + t * NS
            @pl.when(ch < nrch)
            def _():
                pltpu.sync_copy(acc_sh.at[pl.ds(ch * SR, SR)], stage_v)
                pltpu.sync_copy(stage_v, out_hbm.at[cid, pl.ds(ch * SR, SR)])
            return 0
        lax.fori_loop(0, rounds, wb, 0)

    return scat_kernel(y, row3, col3)


_NBLK = 10


def _dinv_from(deg_ref):
    deg = jnp.sum(deg_ref[0] + deg_ref[1], axis=1) * (1.0 / LANES)
    return 1.0 / jnp.sqrt(deg)


def _tc_prescale_matmul(x, wmat, deg2):
    n, d = x.shape
    h = wmat.shape[1]
    r = n // _NBLK

    def body(x_ref, w_ref, deg_ref, y_ref):
        dinv = _dinv_from(deg_ref)
        y_ref[...] = jnp.dot(x_ref[...], w_ref[...],
                             preferred_element_type=jnp.float32,
                             precision=_HIGH) * dinv[:, None]

    return pl.pallas_call(
        body,
        grid=(_NBLK,),
        in_specs=[
            pl.BlockSpec((r, d), lambda i: (i, 0)),
            pl.BlockSpec((d, h), lambda i: (0, 0)),
            pl.BlockSpec((NC, r, LANES), lambda i: (0, i, 0)),
        ],
        out_specs=pl.BlockSpec((r, h), lambda i: (i, 0)),
        out_shape=jax.ShapeDtypeStruct((n, h), jnp.float32),
    )(x, wmat, deg2)


def _h_from(s_ref, deg_ref, b_ref):
    dinv = _dinv_from(deg_ref)
    return (s_ref[0] + s_ref[1]) * dinv[:, None] + b_ref[...]


def _tc_stats(s2, deg2, b):
    _, n, h = s2.shape
    r = n // _NBLK

    def body(s_ref, deg_ref, b_ref, st_ref):
        i = pl.program_id(0)
        hp = _h_from(s_ref, deg_ref, b_ref)

        @pl.when(i == 0)
        def _():
            st_ref[...] = jnp.zeros_like(st_ref)
        st_ref[0:1, :] += jnp.sum(hp, axis=0, keepdims=True)
        st_ref[1:2, :] += jnp.sum(hp * hp, axis=0, keepdims=True)

    return pl.pallas_call(
        body,
        grid=(_NBLK,),
        in_specs=[
            pl.BlockSpec((NC, r, h), lambda i: (0, i, 0)),
            pl.BlockSpec((NC, r, LANES), lambda i: (0, i, 0)),
            pl.BlockSpec((1, h), lambda i: (0, 0)),
        ],
        out_specs=pl.BlockSpec((2, h), lambda i: (0, 0)),
        out_shape=jax.ShapeDtypeStruct((2, h), jnp.float32),
    )(s2, deg2, b)


def _tc_bn_relu_matmul(s2, stats, b, gamma, beta, wmat, deg2):
    _, n, h = s2.shape
    h2 = wmat.shape[1]
    r = n // _NBLK
    inv_n = 1.0 / n

    def body(s_ref, st_ref, b_ref, g_ref, be_ref, w_ref, deg_ref, y_ref):
        dinv = _dinv_from(deg_ref)
        hp = _h_from(s_ref, deg_ref, b_ref)
        mu = st_ref[0:1, :] * inv_n
        var = st_ref[1:2, :] * inv_n - mu * mu
        hn = (hp - mu) / jnp.sqrt(var + 1e-5) * g_ref[...] + be_ref[...]
        hr = jnp.maximum(hn, 0.0)
        y_ref[...] = jnp.dot(hr, w_ref[...],
                             preferred_element_type=jnp.float32,
                             precision=_HIGH) * dinv[:, None]

    return pl.pallas_call(
        body,
        grid=(_NBLK,),
        in_specs=[
            pl.BlockSpec((NC, r, h), lambda i: (0, i, 0)),
            pl.BlockSpec((2, h), lambda i: (0, 0)),
            pl.BlockSpec((1, h), lambda i: (0, 0)),
            pl.BlockSpec((1, h), lambda i: (0, 0)),
            pl.BlockSpec((1, h), lambda i: (0, 0)),
            pl.BlockSpec((h, h2), lambda i: (0, 0)),
            pl.BlockSpec((NC, r, LANES), lambda i: (0, i, 0)),
        ],
        out_specs=pl.BlockSpec((r, h2), lambda i: (i, 0)),
        out_shape=jax.ShapeDtypeStruct((n, h2), jnp.float32),
    )(s2, stats, b, gamma, beta, wmat, deg2)


def _tc_bn_relu_pool_head(s2, stats, b, gamma, beta, batch3d, wf, bf, g,
                          deg2):
    _, n, h = s2.shape
    c = wf.shape[1]
    r = n // _NBLK
    inv_n = 1.0 / n

    def body(s_ref, st_ref, b_ref, g_ref, be_ref, bat_ref, wf_ref, bf_ref,
             deg_ref, o_ref, pooled_ref, counts_ref):
        i = pl.program_id(0)
        hp = _h_from(s_ref, deg_ref, b_ref)
        mu = st_ref[0:1, :] * inv_n
        var = st_ref[1:2, :] * inv_n - mu * mu
        hn = (hp - mu) / jnp.sqrt(var + 1e-5) * g_ref[...] + be_ref[...]
        hr = jnp.maximum(hn, 0.0)

        gids = bat_ref[0]
        gi = lax.broadcasted_iota(jnp.int32, (g, r), 0)
        m = (gi == gids).astype(jnp.float32)

        @pl.when(i == 0)
        def _():
            pooled_ref[...] = jnp.zeros_like(pooled_ref)
            counts_ref[...] = jnp.zeros_like(counts_ref)

        pooled_ref[...] += jnp.dot(m, hr, preferred_element_type=jnp.float32,
                                   precision=_HIGH)
        counts_ref[...] += jnp.sum(m, axis=1, keepdims=True)

        @pl.when(i == _NBLK - 1)
        def _():
            pooled = pooled_ref[...] / jnp.maximum(counts_ref[...], 1.0)
            o_ref[...] = jnp.dot(pooled, wf_ref[...],
                                 preferred_element_type=jnp.float32,
                                 precision=_HIGH) + bf_ref[...]

    return pl.pallas_call(
        body,
        grid=(_NBLK,),
        in_specs=[
            pl.BlockSpec((NC, r, h), lambda i: (0, i, 0)),
            pl.BlockSpec((2, h), lambda i: (0, 0)),
            pl.BlockSpec((1, h), lambda i: (0, 0)),
            pl.BlockSpec((1, h), lambda i: (0, 0)),
            pl.BlockSpec((1, h), lambda i: (0, 0)),
            pl.BlockSpec((1, 1, r), lambda i: (i, 0, 0)),
            pl.BlockSpec((h, c), lambda i: (0, 0)),
            pl.BlockSpec((1, c), lambda i: (0, 0)),
            pl.BlockSpec((NC, r, LANES), lambda i: (0, i, 0)),
        ],
        out_specs=pl.BlockSpec((g, c), lambda i: (0, 0)),
        out_shape=jax.ShapeDtypeStruct((g, c), jnp.float32),
        scratch_shapes=[
            pltpu.VMEM((g, h), jnp.float32),
            pltpu.VMEM((g, 1), jnp.float32),
        ],
    )(s2, stats, b, gamma, beta, batch3d, wf, bf, deg2)


def kernel(x, edge_index, batch, W1, b1, gamma1, beta1, W2, b2, gamma2,
           beta2, Wf, bf):
    n, d = x.shape
    h = W1.shape[1]
    c = Wf.shape[1]
    g = 64

    e = edge_index.shape[1]
    w = NC * NS
    ngrp = 8
    grp = e // (w * B * ngrp)
    row3 = edge_index[0].reshape(w, ngrp, grp, B)
    col3 = edge_index[1].reshape(w, ngrp, grp, B)

    deg2 = _sc_degree(col3, n)
    y1 = _tc_prescale_matmul(x, W1, deg2)
    s1 = _sc_scatter(y1, row3, col3)
    st1 = _tc_stats(s1, deg2, b1.reshape(1, h))
    y2 = _tc_bn_relu_matmul(s1, st1, b1.reshape(1, h), gamma1.reshape(1, h),
                            beta1.reshape(1, h), W2, deg2)
    s2 = _sc_scatter(y2, row3, col3)
    st2 = _tc_stats(s2, deg2, b2.reshape(1, h))
    out = _tc_bn_relu_pool_head(s2, st2, b2.reshape(1, h),
                                gamma2.reshape(1, h), beta2.reshape(1, h),
                                batch.reshape(_NBLK, 1, n // _NBLK),
                                Wf, bf.reshape(1, c), g, deg2)
    return out
